# baseline (speedup 1.0000x reference)
import functools
import math

import jax
import jax.numpy as jnp
from jax.experimental import pallas as pl
from jax.experimental.pallas import tpu as pltpu

_SIGMA = 0.5
_SLOPE = 0.01
_RATIO = 0.5
_HI = jax.lax.Precision.HIGHEST


def _leaky(x):
    return jnp.where(x >= 0, x, _SLOPE * x)


def _bf(x):
    return x.astype(jnp.bfloat16)


def _bfr(x):
    return x.astype(jnp.bfloat16).astype(jnp.float32)


def _conv_body(qT_ref, cpx_ref, cpy_ref, w_ref, kwflat_ref, out_ref,
               kvT_scr, cw_scr, *, n_chunks):
    f32 = jnp.float32
    n = qT_ref.shape[2]
    m = cpx_ref.shape[0]
    c_out = out_ref.shape[1]
    ch = m // n_chunks
    cwf = jnp.dot(_bf(w_ref[...]), _bf(kwflat_ref[...]), preferred_element_type=f32)
    cw_scr[...] = _bf(jnp.reshape(jnp.reshape(cwf, (n, m // n, c_out)), (m, c_out)))
    qTb = _bf(qT_ref[0])
    qx = qT_ref[0, 0:1, :]
    qy = qT_ref[0, 1:2, :]
    q2 = qx * qx + qy * qy
    for c in range(n_chunks):
        cx = cpx_ref[c * ch:(c + 1) * ch, :]
        cy = cpy_ref[c * ch:(c + 1) * ch, :]
        cpslab = jnp.concatenate([cx, cy], axis=1)
        crossT = jnp.dot(_bf(cpslab), qTb, preferred_element_type=f32)
        cp2 = cx * cx + cy * cy
        d2 = jnp.maximum((q2 + cp2) - 2.0 * crossT, 0.0)
        kvT_scr[c * ch:(c + 1) * ch, :] = _bf(jnp.exp(-d2 / (2.0 * _SIGMA ** 2)))
    out_ref[...] = jax.lax.dot_general(kvT_scr[...], cw_scr[...],
                                       (((0,), (0,)), ((), ())),
                                       preferred_element_type=f32)


def _mlp_body(sampled_ref, w_in_ref, pos_ref, g1_ref, bb1_ref, w1_ref, b1_ref,
              g2_ref, bb2_ref, w2w_ref, b2w_ref, w2p_ref, b2p_ref, psel_ref,
              wout_ref, posout_ref, score_ref):
    f32 = jnp.float32
    x = _leaky(sampled_ref[...])
    mu = jnp.mean(x, axis=0, keepdims=True)
    xc = x - mu
    var = jnp.mean(xc * xc, axis=0, keepdims=True)
    x = xc / jnp.sqrt(var + 1e-5) * g1_ref[...] + bb1_ref[...] + w_in_ref[...]
    h = jnp.dot(_bf(x), _bf(w1_ref[...]), preferred_element_type=f32) + b1_ref[...]
    h = _leaky(h)
    mu2 = jnp.mean(h, axis=0, keepdims=True)
    hc = h - mu2
    var2 = jnp.mean(hc * hc, axis=0, keepdims=True)
    h = hc / jnp.sqrt(var2 + 1e-5) * g2_ref[...] + bb2_ref[...]
    dw = jnp.dot(_bf(h), _bf(w2w_ref[...]), preferred_element_type=f32) + b2w_ref[...]
    wout = x + dw
    wout_ref[...] = wout
    dpos = jnp.dot(_bf(h), _bf(w2p_ref[...]), preferred_element_type=f32) + b2p_ref[...]
    posout_ref[...] = pos_ref[...] + dpos
    p = psel_ref[...]
    sc = jnp.dot(_bf(wout), _bf(p), preferred_element_type=f32)
    score_ref[...] = sc / jnp.sqrt(jnp.sum(p * p))


def _select_body(scol_ref, srow_ref, wout_ref, posout_ref,
                 wsel_ref, possel_ref, *, ksel):
    n = scol_ref.shape[1]
    scol = scol_ref[0]
    srow = srow_ref[0]
    ii = jax.lax.broadcasted_iota(jnp.int32, (n, n), 0)
    jj = jax.lax.broadcasted_iota(jnp.int32, (n, n), 1)
    beats = (scol > srow) | ((scol == srow) & (ii < jj))
    rank = jnp.sum(beats.astype(jnp.int32), axis=0, keepdims=True)
    rr = jax.lax.broadcasted_iota(jnp.int32, (ksel, n), 0)
    oht = ((rank == rr) & (rank < ksel)).astype(jnp.float32)
    vals = jnp.sum(oht * srow, axis=1, keepdims=True)
    wgt = jnp.tanh(vals)
    wsel_ref[...] = jnp.dot(oht, wout_ref[...], precision=_HI,
                            preferred_element_type=jnp.float32) * wgt
    possel_ref[...] = jnp.dot(oht, posout_ref[...], precision=_HI,
                              preferred_element_type=jnp.float32)


def _run_conv(positions, weights, kernel_pos, kernel_W, B):
    N, C = weights.shape
    K, pos_dim = kernel_pos.shape
    n_per = N // B
    f32 = jnp.float32
    M = n_per * K
    cpx = (positions[:, 0].reshape(N, 1) + kernel_pos[:, 0].reshape(1, K)).reshape(B * M, 1)
    cpy = (positions[:, 1].reshape(N, 1) + kernel_pos[:, 1].reshape(1, K)).reshape(B * M, 1)
    qT = positions.reshape(B, n_per, pos_dim).transpose(0, 2, 1)
    kwflat = kernel_W.transpose(1, 0, 2).reshape(C, K * C)

    sampled = pl.pallas_call(
        functools.partial(_conv_body, n_chunks=K),
        grid=(B,),
        in_specs=[
            pl.BlockSpec((1, pos_dim, n_per), lambda b: (b, 0, 0)),
            pl.BlockSpec((M, 1), lambda b: (b, 0)),
            pl.BlockSpec((M, 1), lambda b: (b, 0)),
            pl.BlockSpec((n_per, C), lambda b: (b, 0)),
            pl.BlockSpec((C, K * C), lambda b: (0, 0)),
        ],
        out_specs=pl.BlockSpec((n_per, C), lambda b: (b, 0)),
        out_shape=jax.ShapeDtypeStruct((N, C), f32),
        scratch_shapes=[
            pltpu.VMEM((M, n_per), jnp.bfloat16),
            pltpu.VMEM((M, C), jnp.bfloat16),
        ],
    )(qT, cpx, cpy, weights, kwflat)
    return sampled


def _run_mlp(sampled, positions, weights, bn1_g, bn1_b, W1, b1,
             bnm_g, bnm_b, W2, b2, p_sel):
    N, C = weights.shape
    pos_dim = positions.shape[1]
    C_mlp = W1.shape[1]
    f32 = jnp.float32
    wout, posout, score = pl.pallas_call(
        _mlp_body,
        out_shape=[
            jax.ShapeDtypeStruct((N, C), f32),
            jax.ShapeDtypeStruct((N, pos_dim), f32),
            jax.ShapeDtypeStruct((N, 1), f32),
        ],
    )(sampled, weights, positions,
      bn1_g.reshape(1, C), bn1_b.reshape(1, C), W1, b1.reshape(1, C_mlp),
      bnm_g.reshape(1, C_mlp), bnm_b.reshape(1, C_mlp),
      W2[:, pos_dim:], b2[pos_dim:].reshape(1, C),
      W2[:, :pos_dim], b2[:pos_dim].reshape(1, pos_dim),
      p_sel.reshape(C, 1))
    return wout, posout, score


def _run_select(score, wout, posout, B, ksel):
    N, C = wout.shape
    pos_dim = posout.shape[1]
    n_per = N // B
    f32 = jnp.float32
    scol = score.reshape(B, n_per, 1)
    srow = score.reshape(B, 1, n_per)
    wsel, possel = pl.pallas_call(
        functools.partial(_select_body, ksel=ksel),
        grid=(B,),
        in_specs=[
            pl.BlockSpec((1, n_per, 1), lambda b: (b, 0, 0)),
            pl.BlockSpec((1, 1, n_per), lambda b: (b, 0, 0)),
            pl.BlockSpec((n_per, C), lambda b: (b, 0)),
            pl.BlockSpec((n_per, pos_dim), lambda b: (b, 0)),
        ],
        out_specs=[
            pl.BlockSpec((ksel, C), lambda b: (b, 0)),
            pl.BlockSpec((ksel, pos_dim), lambda b: (b, 0)),
        ],
        out_shape=[
            jax.ShapeDtypeStruct((B * ksel, C), f32),
            jax.ShapeDtypeStruct((B * ksel, pos_dim), f32),
        ],
    )(scol, srow, wout, posout)
    return wsel, possel


def kernel(positions, weights, batch, kernel_pos, kernel_W, bn1_g, bn1_b,
           W1, b1, bnm_g, bnm_b, W2, b2, p_sel):
    B = batch.shape[0]
    N, C = weights.shape
    n_per = N // B
    ksel = int(math.ceil(_RATIO * n_per))
    sampled = _run_conv(positions, weights, kernel_pos, kernel_W, B)
    wout, posout, score = _run_mlp(sampled, positions, weights, bn1_g, bn1_b,
                                   W1, b1, bnm_g, bnm_b, W2, b2, p_sel)
    wsel, possel = _run_select(score, wout, posout, B, ksel)
    new_batch = jnp.full((B,), ksel, dtype=batch.dtype)
    return possel, wsel, new_batch
